# baseline (speedup 1.0000x reference)
import jax
import jax.numpy as jnp
from jax.experimental import pallas as pl
from jax.experimental.pallas import tpu as pltpu


def _rup(x, m):
    return (x + m - 1) // m * m


def _fold_bn(gamma, beta, mean, var, eps=1e-5):
    scale = gamma / jnp.sqrt(var + eps)
    return scale, beta - mean * scale


OFF = 128


def _make_body(C, P0, NB, MARGIN, L, W, PLANE):
    def _build_t(src, km_ref, t_ref):
        t_ref[0:C, :] = pltpu.roll(src, shift=W, axis=1) * km_ref[0:1, :]
        t_ref[C:2 * C, :] = src
        t_ref[2 * C:3 * C, :] = \
            pltpu.roll(src, shift=(-W) % L, axis=1) * km_ref[1:2, :]

    def _conv(t_ref, b9_ref, w_ref, y_ref):
        for kd in range(3):
            s = MARGIN - OFF + (kd - 1) * PLANE
            d = jnp.dot(w_ref[kd], t_ref[:, s:s + NB],
                        preferred_element_type=jnp.float32)
            if kd == 0:
                y_ref[...] = d
            else:
                y_ref[...] += d

    def _combine(y_ref, wm_ref):
        return (y_ref[0:C, OFF - 1:OFF - 1 + P0] * wm_ref[0:1, :] +
                y_ref[C:2 * C, OFF:OFF + P0] +
                y_ref[2 * C:3 * C, OFF + 1:OFF + 1 + P0] * wm_ref[1:2, :])

    def body(x_ref, wa_ref, ba_ref, wb_ref, bb_ref, km_ref, wm_ref, o_ref,
             t_ref, y_ref, f_ref):
        for b in range(2):
            t_r, y_r, f_r = t_ref.at[b], y_ref.at[b], f_ref.at[b]
            f_r[:, 0:MARGIN] = jnp.zeros((C, MARGIN), jnp.bfloat16)
            f_r[:, MARGIN + P0:L] = jnp.zeros((C, L - MARGIN - P0), jnp.bfloat16)
            f_r[:, MARGIN:MARGIN + P0] = x_ref[b].astype(jnp.bfloat16)
            _build_t(f_r[...], km_ref, t_r)
            _conv(t_r, None, wa_ref, y_r)
            h = jnp.maximum(_combine(y_r, wm_ref) + ba_ref[...], 0.0)
            f_r[:, MARGIN:MARGIN + P0] = h.astype(jnp.bfloat16)

            _build_t(f_r[...], km_ref, t_r)
            _conv(t_r, None, wb_ref, y_r)
            o_ref[b] = jnp.maximum(
                _combine(y_r, wm_ref) + bb_ref[...] + x_ref[b], 0.0)

    return body


def kernel(x, w1, g1, b1, m1, v1, w2, g2, b2, m2, v2):
    N, C, D, H, W = x.shape
    P0 = D * H * W
    PLANE = H * W
    MARGIN = _rup(OFF + PLANE, 128)
    L = MARGIN + P0 + MARGIN
    NB = P0 + 2 * OFF
    assert PLANE % 128 == 0 and MARGIN >= OFF + PLANE

    scale1, bias1 = _fold_bn(g1, b1, m1, v1)
    scale2, bias2 = _fold_bn(g2, b2, m2, v2)
    w1s = w1 * scale1[:, None, None, None, None]
    w2s = w2 * scale2[:, None, None, None, None]
    wa = jnp.transpose(w1s, (2, 4, 0, 3, 1)).reshape(3, 3 * C, 3 * C)
    wb = jnp.transpose(w2s, (2, 4, 0, 3, 1)).reshape(3, 3 * C, 3 * C)
    wa = wa.astype(jnp.bfloat16)
    wb = wb.astype(jnp.bfloat16)
    ba = bias1.reshape(C, 1).astype(jnp.float32)
    bb = bias2.reshape(C, 1).astype(jnp.float32)

    cp = jnp.arange(L)
    h0 = ((cp - OFF) // W) % H
    kmask = jnp.stack([(h0 >= 1), (h0 <= H - 2)]
                      + [jnp.zeros((L,), bool)] * 6).astype(jnp.bfloat16)
    wj = jnp.arange(P0) % W
    wmask = jnp.stack([(wj >= 1), (wj <= W - 2)]
                      + [jnp.zeros((P0,), bool)] * 6).astype(jnp.float32)

    body = _make_body(C, P0, NB, MARGIN, L, W, PLANE)

    x_flat = x.reshape(N, C, P0)

    flops = 2 * (2 * 27 * C * C * P0) * N
    bytes_accessed = int(4 * x_flat.size + 4 * (N * C * P0)
                         + 2 * (wa.size + wb.size))

    out_flat = pl.pallas_call(
        body,
        out_shape=jax.ShapeDtypeStruct((N, C, P0), jnp.float32),
        grid=(N // 2,),
        in_specs=[
            pl.BlockSpec((2, C, P0), lambda n: (n, 0, 0)),
            pl.BlockSpec((3, 3 * C, 3 * C), lambda n: (0, 0, 0)),
            pl.BlockSpec((C, 1), lambda n: (0, 0)),
            pl.BlockSpec((3, 3 * C, 3 * C), lambda n: (0, 0, 0)),
            pl.BlockSpec((C, 1), lambda n: (0, 0)),
            pl.BlockSpec((8, L), lambda n: (0, 0)),
            pl.BlockSpec((8, P0), lambda n: (0, 0)),
        ],
        out_specs=pl.BlockSpec((2, C, P0), lambda n: (n, 0, 0)),
        scratch_shapes=[
            pltpu.VMEM((2, 3 * C, L), jnp.bfloat16),
            pltpu.VMEM((2, 3 * C, NB), jnp.float32),
            pltpu.VMEM((2, C, L), jnp.bfloat16),
        ],
        compiler_params=pltpu.CompilerParams(
            dimension_semantics=("parallel",),
            vmem_limit_bytes=64 * 1024 * 1024),
        cost_estimate=pl.CostEstimate(
            flops=flops, transcendentals=0, bytes_accessed=bytes_accessed),
    )(x_flat, wa, ba, wb, bb, kmask, wmask)

    return out_flat.reshape(N, C, D, H, W)


# speedup vs baseline: 1.1451x; 1.1451x over previous
import jax
import jax.numpy as jnp
from jax.experimental import pallas as pl
from jax.experimental.pallas import tpu as pltpu


def _rup(x, m):
    return (x + m - 1) // m * m


def _fold_bn(gamma, beta, mean, var, eps=1e-5):
    scale = gamma / jnp.sqrt(var + eps)
    return scale, beta - mean * scale


OFF = 128


def _make_body(C, P0, NB, MARGIN, L, W, PLANE):
    def _build_t(src, km_ref, t_ref):
        t_ref[0:C, :] = pltpu.roll(src, shift=W, axis=1) * km_ref[0:1, :]
        t_ref[C:2 * C, :] = src
        t_ref[2 * C:3 * C, :] = \
            pltpu.roll(src, shift=(-W) % L, axis=1) * km_ref[1:2, :]

    def _conv(t_ref, b9_ref, w_ref, y_ref):
        for kd in range(3):
            s = MARGIN - OFF + (kd - 1) * PLANE
            for kh in range(3):
                g = kd * 3 + kh
                b9_ref[g * C:(g + 1) * C, :] = \
                    t_ref[kh * C:(kh + 1) * C, s:s + NB]
        y_ref[...] = jnp.dot(w_ref[...], b9_ref[...],
                             preferred_element_type=jnp.float32)

    def _combine(y_ref, wm_ref):
        return (y_ref[0:C, OFF - 1:OFF - 1 + P0] * wm_ref[0:1, :] +
                y_ref[C:2 * C, OFF:OFF + P0] +
                y_ref[2 * C:3 * C, OFF + 1:OFF + 1 + P0] * wm_ref[1:2, :])

    def body(x_ref, wa_ref, ba_ref, wb_ref, bb_ref, km_ref, wm_ref, o_ref,
             t_ref, b9_ref, y_ref, f_ref):
        for b in range(2):
            t_r, b9_r, y_r, f_r = t_ref.at[b], b9_ref.at[b], y_ref.at[b], f_ref.at[b]
            f_r[:, 0:MARGIN] = jnp.zeros((C, MARGIN), jnp.bfloat16)
            f_r[:, MARGIN + P0:L] = jnp.zeros((C, L - MARGIN - P0), jnp.bfloat16)
            f_r[:, MARGIN:MARGIN + P0] = x_ref[b].astype(jnp.bfloat16)
            _build_t(f_r[...], km_ref, t_r)
            _conv(t_r, b9_r, wa_ref, y_r)
            h = jnp.maximum(_combine(y_r, wm_ref) + ba_ref[...], 0.0)
            f_r[:, MARGIN:MARGIN + P0] = h.astype(jnp.bfloat16)

            _build_t(f_r[...], km_ref, t_r)
            _conv(t_r, b9_r, wb_ref, y_r)
            o_ref[b] = jnp.maximum(
                _combine(y_r, wm_ref) + bb_ref[...] + x_ref[b], 0.0)

    return body


def kernel(x, w1, g1, b1, m1, v1, w2, g2, b2, m2, v2):
    N, C, D, H, W = x.shape
    P0 = D * H * W
    PLANE = H * W
    MARGIN = _rup(OFF + PLANE, 128)
    L = MARGIN + P0 + MARGIN
    NB = P0 + 2 * OFF
    assert PLANE % 128 == 0 and MARGIN >= OFF + PLANE

    scale1, bias1 = _fold_bn(g1, b1, m1, v1)
    scale2, bias2 = _fold_bn(g2, b2, m2, v2)
    w1s = w1 * scale1[:, None, None, None, None]
    w2s = w2 * scale2[:, None, None, None, None]
    wa = jnp.transpose(w1s, (4, 0, 2, 3, 1)).reshape(3 * C, 9 * C)
    wb = jnp.transpose(w2s, (4, 0, 2, 3, 1)).reshape(3 * C, 9 * C)
    wa = wa.astype(jnp.bfloat16)
    wb = wb.astype(jnp.bfloat16)
    ba = bias1.reshape(C, 1).astype(jnp.float32)
    bb = bias2.reshape(C, 1).astype(jnp.float32)

    cp = jnp.arange(L)
    h0 = ((cp - OFF) // W) % H
    kmask = jnp.stack([(h0 >= 1), (h0 <= H - 2)]
                      + [jnp.zeros((L,), bool)] * 6).astype(jnp.bfloat16)
    wj = jnp.arange(P0) % W
    wmask = jnp.stack([(wj >= 1), (wj <= W - 2)]
                      + [jnp.zeros((P0,), bool)] * 6).astype(jnp.float32)

    body = _make_body(C, P0, NB, MARGIN, L, W, PLANE)

    x_flat = x.reshape(N, C, P0)

    flops = 2 * (2 * 27 * C * C * P0) * N
    bytes_accessed = int(4 * x_flat.size + 4 * (N * C * P0)
                         + 2 * (wa.size + wb.size))

    out_flat = pl.pallas_call(
        body,
        out_shape=jax.ShapeDtypeStruct((N, C, P0), jnp.float32),
        grid=(N // 2,),
        in_specs=[
            pl.BlockSpec((2, C, P0), lambda n: (n, 0, 0)),
            pl.BlockSpec((3 * C, 9 * C), lambda n: (0, 0)),
            pl.BlockSpec((C, 1), lambda n: (0, 0)),
            pl.BlockSpec((3 * C, 9 * C), lambda n: (0, 0)),
            pl.BlockSpec((C, 1), lambda n: (0, 0)),
            pl.BlockSpec((8, L), lambda n: (0, 0)),
            pl.BlockSpec((8, P0), lambda n: (0, 0)),
        ],
        out_specs=pl.BlockSpec((2, C, P0), lambda n: (n, 0, 0)),
        scratch_shapes=[
            pltpu.VMEM((2, 3 * C, L), jnp.bfloat16),
            pltpu.VMEM((2, 9 * C, NB), jnp.bfloat16),
            pltpu.VMEM((2, 3 * C, NB), jnp.float32),
            pltpu.VMEM((2, C, L), jnp.bfloat16),
        ],
        compiler_params=pltpu.CompilerParams(
            dimension_semantics=("parallel",),
            vmem_limit_bytes=64 * 1024 * 1024),
        cost_estimate=pl.CostEstimate(
            flops=flops, transcendentals=0, bytes_accessed=bytes_accessed),
    )(x_flat, wa, ba, wb, bb, kmask, wmask)

    return out_flat.reshape(N, C, D, H, W)
